# baseline (speedup 1.0000x reference)
import functools

import jax
import jax.numpy as jnp
from jax import lax
from jax.experimental import pallas as pl
from jax.experimental.pallas import tpu as pltpu
from jax.experimental.pallas import tpu_sc as plsc

_NC = 2
_NS = 16
_L = 16
_NW = _NC * _NS
_CHUNK = 128


def _cdiv(a, b):
  return (a + b - 1) // b


def _proj(x, w1, b1):
  n, d = x.shape
  blk = 1000

  def body(x_ref, w1_ref, b1_ref, a_ref, b_ref):
    xa = x_ref[...]
    a_ref[...] = (
        jnp.dot(xa, w1_ref[0:d, :], preferred_element_type=jnp.float32)
        + b1_ref[...]
    )
    b_ref[...] = jnp.dot(xa, w1_ref[d : 2 * d, :],
                         preferred_element_type=jnp.float32)

  return pl.pallas_call(
      body,
      grid=(n // blk,),
      in_specs=[
          pl.BlockSpec((blk, d), lambda i: (i, 0)),
          pl.BlockSpec((2 * d, d), lambda i: (0, 0)),
          pl.BlockSpec((1, d), lambda i: (0, 0)),
      ],
      out_specs=[
          pl.BlockSpec((blk, d), lambda i: (i, 0)),
          pl.BlockSpec((blk, d), lambda i: (i, 0)),
      ],
      out_shape=[
          jax.ShapeDtypeStruct((n, d), jnp.float32),
          jax.ShapeDtypeStruct((n, d), jnp.float32),
      ],
  )(x, w1, b1.reshape(1, d))


def _sc_logits(a, b, row, col, w2):
  n, d = a.shape
  e_total = row.shape[0]
  nj = d // _L
  n_chunks = e_total // _CHUNK
  k_iters = _cdiv(n_chunks, _NW)
  mesh = plsc.VectorSubcoreMesh(core_axis_name="c", subcore_axis_name="s")

  @functools.partial(
      pl.kernel,
      mesh=mesh,
      compiler_params=pltpu.CompilerParams(needs_layout_passes=False),
      out_type=jax.ShapeDtypeStruct((e_total,), jnp.float32),
      scratch_types=[
          pltpu.VMEM((_CHUNK,), jnp.int32),
          pltpu.VMEM((_CHUNK,), jnp.int32),
          pltpu.VMEM((_CHUNK, d), jnp.float32),
          pltpu.VMEM((_CHUNK, d), jnp.float32),
          pltpu.VMEM((_CHUNK * _L,), jnp.float32),
          pltpu.VMEM((_CHUNK,), jnp.float32),
          pltpu.VMEM((d,), jnp.float32),
          pltpu.SemaphoreType.DMA,
          pltpu.SemaphoreType.DMA,
      ],
  )
  def k(row_hbm, col_hbm, a_hbm, b_hbm, w2_hbm, out_hbm, rowv, colv, arows,
        brows, part, logitv, w2v, sem_a, sem_b):
    cid = lax.axis_index("c")
    sid = lax.axis_index("s")
    wid = sid * _NC + cid
    pltpu.sync_copy(w2_hbm, w2v)
    w2b = [w2v[pl.ds(j * _L, _L)] for j in range(nj)]
    iota = lax.iota(jnp.int32, _L)

    def chunk_body(kk, carry):
      c = wid + _NW * kk

      @pl.when(c < n_chunks)
      def _():
        base = c * _CHUNK
        pltpu.sync_copy(row_hbm.at[pl.ds(base, _CHUNK)], rowv)
        pltpu.sync_copy(col_hbm.at[pl.ds(base, _CHUNK)], colv)
        cp_a = pltpu.async_copy(a_hbm.at[rowv], arows, sem_a)
        cp_b = pltpu.async_copy(b_hbm.at[colv], brows, sem_b)
        cp_a.wait()
        cp_b.wait()

        def edge_body(e, acc0):
          acc = jnp.zeros((_L,), jnp.float32)
          for j in range(nj):
            av = arows[e, pl.ds(j * _L, _L)]
            bv = brows[e, pl.ds(j * _L, _L)]
            acc = acc + jnp.maximum(av + bv, 0.0) * w2b[j]
          part[pl.ds(e * _L, _L)] = acc
          return acc0

        lax.fori_loop(0, _CHUNK, edge_body, 0)

        def grp_body(g, acc0):
          ridx = (g * _L + iota) * _L
          tot = jnp.zeros((_L,), jnp.float32)
          for j in range(_L):
            tot = tot + plsc.load_gather(part, [ridx + j])
          logitv[pl.ds(g * _L, _L)] = tot
          return acc0

        lax.fori_loop(0, _CHUNK // _L, grp_body, 0)
        pltpu.sync_copy(logitv, out_hbm.at[pl.ds(base, _CHUNK)])

      return carry

    lax.fori_loop(0, k_iters, chunk_body, 0)

  return k(row, col, a, b, w2)


def _softmax(l2d):
  def body(l_ref, w_ref):
    l = l_ref[...]
    m = jnp.max(l)
    ex = jnp.exp(l - m)
    w_ref[...] = ex * (1.0 / jnp.sum(ex))

  return pl.pallas_call(
      body, out_shape=jax.ShapeDtypeStruct(l2d.shape, jnp.float32)
  )(l2d)


def _sc_scatter(x, row, col, w):
  n, d = x.shape
  e_total = row.shape[0]
  ck = 16
  hw = d // 2
  nj = d // _L
  nh = hw // _L
  ept = e_total // _NS
  n_ch = ept // ck
  half = n // _NC
  seg = ((half + 64 + (_NS * 8) - 1) // (_NS * 8)) * (_NS * 8)
  acc_rows = 2 * seg
  rows_per_tile = acc_rows // _NS
  n_wb = half // 8
  mesh = plsc.VectorSubcoreMesh(core_axis_name="c", subcore_axis_name="s")

  @functools.partial(
      pl.kernel,
      mesh=mesh,
      compiler_params=pltpu.CompilerParams(needs_layout_passes=False),
      out_type=jax.ShapeDtypeStruct((n, d), jnp.float32),
      scratch_types=[
          pltpu.VMEM((ept,), jnp.int32),
          pltpu.VMEM((ept,), jnp.int32),
          pltpu.VMEM((ept,), jnp.float32),
          pltpu.VMEM((4, ck), jnp.int32),
          pltpu.VMEM((2, ck, d), jnp.float32),
          pltpu.VMEM((2, 2 * ck, hw), jnp.float32),
          pltpu.VMEM_SHARED((acc_rows, hw), jnp.float32),
          pltpu.SemaphoreType.DMA,
          pltpu.SemaphoreType.DMA,
          pltpu.SemaphoreType.DMA,
          pltpu.SemaphoreType.DMA,
      ],
  )
  def k(x_hbm, row_hbm, col_hbm, w_hbm, out_hbm, rowa, cola, wa, idx2, xg,
        xs, acc, sem_g0, sem_g1, sem_s0, sem_s1):
    cid = lax.axis_index("c")
    sid = lax.axis_index("s")
    nbase = cid * half
    tbase = sid * ept
    sem_g = (sem_g0, sem_g1)
    sem_s = (sem_s0, sem_s1)

    zv = jnp.zeros((_L,), jnp.float32)

    def zrow(e, carry):
      for j in range(nh):
        xs[0, e, pl.ds(j * _L, _L)] = zv
      return carry

    lax.fori_loop(0, 2 * ck, zrow, 0)
    rbase = sid * rows_per_tile
    for off in range(0, rows_per_tile, 2 * ck):
      sz = min(2 * ck, rows_per_tile - off)
      pltpu.sync_copy(xs.at[0, pl.ds(0, sz)], acc.at[pl.ds(rbase + off, sz)])

    pltpu.sync_copy(row_hbm.at[pl.ds(tbase, ept)], rowa)
    pltpu.sync_copy(col_hbm.at[pl.ds(tbase, ept)], cola)
    pltpu.sync_copy(w_hbm.at[pl.ds(tbase, ept)], wa)
    plsc.subcore_barrier()

    def issue_gather(c, par, sem):
      return pltpu.async_copy(
          x_hbm.at[rowa.at[pl.ds(c * ck, ck)]], xg.at[par], sem)

    def process(c, par, k2):
      @pl.when(k2 > 0)
      def _():
        for _h in range(2):
          pltpu.make_async_copy(
              xs.at[par, pl.ds(0, ck)], acc.at[idx2.at[2 * par]],
              sem_s[par]).wait()

      def grp_body(g, acc0):
        cv = cola[pl.ds(c * ck + g * _L, _L)]
        lv = cv - nbase
        ok = (lv >= 0) & (lv < half)
        li = jnp.where(ok, lv, half + (cv & 63))
        idx2[2 * par, pl.ds(g * _L, _L)] = li
        idx2[2 * par + 1, pl.ds(g * _L, _L)] = li + seg
        return acc0

      lax.fori_loop(0, ck // _L, grp_body, 0)

      def scale(e, acc0):
        wb = plsc.load_gather(wa, [jnp.full((_L,), c * ck, jnp.int32) + e])
        for j in range(nh):
          xs[par, e, pl.ds(j * _L, _L)] = xg[par, e, pl.ds(j * _L, _L)] * wb
        for j in range(nh):
          xs[par, ck + e, pl.ds(j * _L, _L)] = (
              xg[par, e, pl.ds((nh + j) * _L, _L)] * wb)
        return acc0

      lax.fori_loop(0, ck, scale, 0)

      for h in range(2):
        pltpu.async_copy(xs.at[par, pl.ds(h * ck, ck)],
                         acc.at[idx2.at[2 * par + h]], sem_s[par], add=True)

    cp0 = issue_gather(0, 0, sem_g0)
    cp0.wait()

    def outer(k2, carry):
      c0 = 2 * k2
      issue_gather(c0 + 1, 1, sem_g1)
      process(c0, 0, k2)
      pltpu.make_async_copy(
          x_hbm.at[rowa.at[pl.ds(0, ck)]], xg.at[1], sem_g1).wait()

      @pl.when(k2 < n_ch // 2 - 1)
      def _():
        issue_gather(c0 + 2, 0, sem_g0)

      process(c0 + 1, 1, k2)

      @pl.when(k2 < n_ch // 2 - 1)
      def _():
        pltpu.make_async_copy(
            x_hbm.at[rowa.at[pl.ds(0, ck)]], xg.at[0], sem_g0).wait()

      return carry

    lax.fori_loop(0, n_ch // 2, outer, 0)
    if n_ch % 2 == 1:
      issue_gather(n_ch - 1, 0, sem_g0)
      pltpu.make_async_copy(
          x_hbm.at[rowa.at[pl.ds(0, ck)]], xg.at[0], sem_g0).wait()
      process(n_ch - 1, 0, 1)
    for par in range(2):
      for _h in range(2):
        pltpu.make_async_copy(
            xs.at[par, pl.ds(0, ck)], acc.at[idx2.at[2 * par]],
            sem_s[par]).wait()
    plsc.subcore_barrier()

    def wb_body(k2, carry):
      c8 = sid + _NS * k2

      @pl.when(c8 < n_wb)
      def _():
        rb = c8 * 8
        pltpu.sync_copy(acc.at[pl.ds(rb, 8)],
                        out_hbm.at[pl.ds(nbase + rb, 8), pl.ds(0, hw)])
        pltpu.sync_copy(acc.at[pl.ds(seg + rb, 8)],
                        out_hbm.at[pl.ds(nbase + rb, 8), pl.ds(hw, hw)])
      return carry

    lax.fori_loop(0, _cdiv(n_wb, _NS), wb_body, 0)

  return k(x, row, col, w)


def kernel(x, edge_index, W1, b1, W2, b2):
  n, d = x.shape
  e_total = edge_index.shape[1]
  row = edge_index[0]
  col = edge_index[1]
  a, b = _proj(x, W1, b1)
  logits = _sc_logits(a, b, row, col, W2.reshape(d))
  w = _softmax(logits.reshape(e_total // 128, 128)).reshape(e_total)
  out = _sc_scatter(x, row, col, w)
  return out, w.reshape(e_total, 1)


# speedup vs baseline: 1.1248x; 1.1248x over previous
import functools

import jax
import jax.numpy as jnp
from jax import lax
from jax.experimental import pallas as pl
from jax.experimental.pallas import tpu as pltpu
from jax.experimental.pallas import tpu_sc as plsc

_NC = 2
_NS = 16
_L = 16
_NW = _NC * _NS
_CHUNK = 128


def _cdiv(a, b):
  return (a + b - 1) // b


def _proj(x, w1, b1):
  n, d = x.shape
  blk = 1000

  def body(x_ref, w1_ref, b1_ref, a_ref, b_ref):
    xa = x_ref[...]
    a_ref[...] = (
        jnp.dot(xa, w1_ref[0:d, :], preferred_element_type=jnp.float32)
        + b1_ref[...]
    )
    b_ref[...] = jnp.dot(xa, w1_ref[d : 2 * d, :],
                         preferred_element_type=jnp.float32)

  return pl.pallas_call(
      body,
      grid=(n // blk,),
      in_specs=[
          pl.BlockSpec((blk, d), lambda i: (i, 0)),
          pl.BlockSpec((2 * d, d), lambda i: (0, 0)),
          pl.BlockSpec((1, d), lambda i: (0, 0)),
      ],
      out_specs=[
          pl.BlockSpec((blk, d), lambda i: (i, 0)),
          pl.BlockSpec((blk, d), lambda i: (i, 0)),
      ],
      out_shape=[
          jax.ShapeDtypeStruct((n, d), jnp.float32),
          jax.ShapeDtypeStruct((n, d), jnp.float32),
      ],
  )(x, w1, b1.reshape(1, d))


def _sc_logits(a, b, row, col, w2):
  n, d = a.shape
  e_total = row.shape[0]
  nj = d // _L
  ck = 40
  ckp = 48
  ept = e_total // _NW
  n_ch = ept // ck
  mesh = plsc.VectorSubcoreMesh(core_axis_name="c", subcore_axis_name="s")

  @functools.partial(
      pl.kernel,
      mesh=mesh,
      compiler_params=pltpu.CompilerParams(needs_layout_passes=False),
      out_type=jax.ShapeDtypeStruct((e_total,), jnp.float32),
      scratch_types=[
          pltpu.VMEM((ept,), jnp.int32),
          pltpu.VMEM((ept,), jnp.int32),
          pltpu.VMEM((2, ckp, d), jnp.float32),
          pltpu.VMEM((2, ckp, d), jnp.float32),
          pltpu.VMEM((ckp * _L,), jnp.float32),
          pltpu.VMEM((ckp,), jnp.float32),
          pltpu.VMEM((d,), jnp.float32),
          pltpu.SemaphoreType.DMA,
          pltpu.SemaphoreType.DMA,
          pltpu.SemaphoreType.DMA,
          pltpu.SemaphoreType.DMA,
      ],
  )
  def k(row_hbm, col_hbm, a_hbm, b_hbm, w2_hbm, out_hbm, rowa, cola, arows,
        brows, part, logitv, w2v, sem_a0, sem_a1, sem_b0, sem_b1):
    cid = lax.axis_index("c")
    sid = lax.axis_index("s")
    wid = sid * _NC + cid
    tbase = wid * ept
    sem_a = (sem_a0, sem_a1)
    sem_b = (sem_b0, sem_b1)
    pltpu.sync_copy(w2_hbm, w2v)
    pltpu.sync_copy(row_hbm.at[pl.ds(tbase, ept)], rowa)
    pltpu.sync_copy(col_hbm.at[pl.ds(tbase, ept)], cola)
    w2b = [w2v[pl.ds(j * _L, _L)] for j in range(nj)]
    iota = lax.iota(jnp.int32, _L)

    def issue(c, par):
      pltpu.async_copy(a_hbm.at[rowa.at[pl.ds(c * ck, ck)]],
                       arows.at[par, pl.ds(0, ck)], sem_a[par])
      pltpu.async_copy(b_hbm.at[cola.at[pl.ds(c * ck, ck)]],
                       brows.at[par, pl.ds(0, ck)], sem_b[par])

    def wait_g(par):
      pltpu.make_async_copy(a_hbm.at[rowa.at[pl.ds(0, ck)]],
                            arows.at[par, pl.ds(0, ck)], sem_a[par]).wait()
      pltpu.make_async_copy(b_hbm.at[cola.at[pl.ds(0, ck)]],
                            brows.at[par, pl.ds(0, ck)], sem_b[par]).wait()

    def process(c, par):
      def edge_body(e, acc0):
        acc = jnp.zeros((_L,), jnp.float32)
        for j in range(nj):
          av = arows[par, e, pl.ds(j * _L, _L)]
          bv = brows[par, e, pl.ds(j * _L, _L)]
          acc = acc + jnp.maximum(av + bv, 0.0) * w2b[j]
        part[pl.ds(e * _L, _L)] = acc
        return acc0

      lax.fori_loop(0, ckp, edge_body, 0)

      def grp_body(g, acc0):
        ridx = (g * _L + iota) * _L
        tot = jnp.zeros((_L,), jnp.float32)
        for j in range(_L):
          tot = tot + plsc.load_gather(part, [ridx + j])
        logitv[pl.ds(g * _L, _L)] = tot
        return acc0

      lax.fori_loop(0, ckp // _L, grp_body, 0)
      pltpu.sync_copy(logitv.at[pl.ds(0, ck)],
                      out_hbm.at[pl.ds(tbase + c * ck, ck)])

    issue(0, 0)
    wait_g(0)

    def outer(k2, carry):
      c0 = 2 * k2
      issue(c0 + 1, 1)
      process(c0, 0)
      wait_g(1)

      @pl.when(k2 < n_ch // 2 - 1)
      def _():
        issue(c0 + 2, 0)

      process(c0 + 1, 1)

      @pl.when(k2 < n_ch // 2 - 1)
      def _():
        wait_g(0)

      return carry

    lax.fori_loop(0, n_ch // 2, outer, 0)
    if n_ch % 2 == 1:
      issue(n_ch - 1, 0)
      wait_g(0)
      process(n_ch - 1, 0)

  return k(row, col, a, b, w2)


def _softmax(l2d):
  def body(l_ref, w_ref):
    l = l_ref[...]
    m = jnp.max(l)
    ex = jnp.exp(l - m)
    w_ref[...] = ex * (1.0 / jnp.sum(ex))

  return pl.pallas_call(
      body, out_shape=jax.ShapeDtypeStruct(l2d.shape, jnp.float32)
  )(l2d)


def _sc_scatter(x, row, col, w):
  n, d = x.shape
  e_total = row.shape[0]
  ck = 16
  hw = d // 2
  nj = d // _L
  nh = hw // _L
  ept = e_total // _NS
  n_ch = ept // ck
  half = n // _NC
  seg = ((half + 64 + (_NS * 8) - 1) // (_NS * 8)) * (_NS * 8)
  acc_rows = 2 * seg
  rows_per_tile = acc_rows // _NS
  n_wb = half // 8
  mesh = plsc.VectorSubcoreMesh(core_axis_name="c", subcore_axis_name="s")

  @functools.partial(
      pl.kernel,
      mesh=mesh,
      compiler_params=pltpu.CompilerParams(needs_layout_passes=False),
      out_type=jax.ShapeDtypeStruct((n, d), jnp.float32),
      scratch_types=[
          pltpu.VMEM((ept,), jnp.int32),
          pltpu.VMEM((ept,), jnp.int32),
          pltpu.VMEM((ept,), jnp.float32),
          pltpu.VMEM((4, ck), jnp.int32),
          pltpu.VMEM((2, ck, d), jnp.float32),
          pltpu.VMEM((2, 2 * ck, hw), jnp.float32),
          pltpu.VMEM_SHARED((acc_rows, hw), jnp.float32),
          pltpu.SemaphoreType.DMA,
          pltpu.SemaphoreType.DMA,
          pltpu.SemaphoreType.DMA,
          pltpu.SemaphoreType.DMA,
      ],
  )
  def k(x_hbm, row_hbm, col_hbm, w_hbm, out_hbm, rowa, cola, wa, idx2, xg,
        xs, acc, sem_g0, sem_g1, sem_s0, sem_s1):
    cid = lax.axis_index("c")
    sid = lax.axis_index("s")
    nbase = cid * half
    tbase = sid * ept
    sem_g = (sem_g0, sem_g1)
    sem_s = (sem_s0, sem_s1)

    zv = jnp.zeros((_L,), jnp.float32)

    def zrow(e, carry):
      for j in range(nh):
        xs[0, e, pl.ds(j * _L, _L)] = zv
      return carry

    lax.fori_loop(0, 2 * ck, zrow, 0)
    rbase = sid * rows_per_tile
    for off in range(0, rows_per_tile, 2 * ck):
      sz = min(2 * ck, rows_per_tile - off)
      pltpu.sync_copy(xs.at[0, pl.ds(0, sz)], acc.at[pl.ds(rbase + off, sz)])

    pltpu.sync_copy(row_hbm.at[pl.ds(tbase, ept)], rowa)
    pltpu.sync_copy(col_hbm.at[pl.ds(tbase, ept)], cola)
    pltpu.sync_copy(w_hbm.at[pl.ds(tbase, ept)], wa)
    plsc.subcore_barrier()

    def issue_gather(c, par, sem):
      return pltpu.async_copy(
          x_hbm.at[rowa.at[pl.ds(c * ck, ck)]], xg.at[par], sem)

    def process(c, par, k2):
      @pl.when(k2 > 0)
      def _():
        for _h in range(2):
          pltpu.make_async_copy(
              xs.at[par, pl.ds(0, ck)], acc.at[idx2.at[2 * par]],
              sem_s[par]).wait()

      def grp_body(g, acc0):
        cv = cola[pl.ds(c * ck + g * _L, _L)]
        lv = cv - nbase
        ok = (lv >= 0) & (lv < half)
        li = jnp.where(ok, lv, half + (cv & 63))
        idx2[2 * par, pl.ds(g * _L, _L)] = li
        idx2[2 * par + 1, pl.ds(g * _L, _L)] = li + seg
        return acc0

      lax.fori_loop(0, ck // _L, grp_body, 0)

      def scale(e, acc0):
        wb = plsc.load_gather(wa, [jnp.full((_L,), c * ck, jnp.int32) + e])
        for j in range(nh):
          xs[par, e, pl.ds(j * _L, _L)] = xg[par, e, pl.ds(j * _L, _L)] * wb
        for j in range(nh):
          xs[par, ck + e, pl.ds(j * _L, _L)] = (
              xg[par, e, pl.ds((nh + j) * _L, _L)] * wb)
        return acc0

      lax.fori_loop(0, ck, scale, 0)

      for h in range(2):
        pltpu.async_copy(xs.at[par, pl.ds(h * ck, ck)],
                         acc.at[idx2.at[2 * par + h]], sem_s[par], add=True)

    cp0 = issue_gather(0, 0, sem_g0)
    cp0.wait()

    def outer(k2, carry):
      c0 = 2 * k2
      issue_gather(c0 + 1, 1, sem_g1)
      process(c0, 0, k2)
      pltpu.make_async_copy(
          x_hbm.at[rowa.at[pl.ds(0, ck)]], xg.at[1], sem_g1).wait()

      @pl.when(k2 < n_ch // 2 - 1)
      def _():
        issue_gather(c0 + 2, 0, sem_g0)

      process(c0 + 1, 1, k2)

      @pl.when(k2 < n_ch // 2 - 1)
      def _():
        pltpu.make_async_copy(
            x_hbm.at[rowa.at[pl.ds(0, ck)]], xg.at[0], sem_g0).wait()

      return carry

    lax.fori_loop(0, n_ch // 2, outer, 0)
    if n_ch % 2 == 1:
      issue_gather(n_ch - 1, 0, sem_g0)
      pltpu.make_async_copy(
          x_hbm.at[rowa.at[pl.ds(0, ck)]], xg.at[0], sem_g0).wait()
      process(n_ch - 1, 0, 1)
    for par in range(2):
      for _h in range(2):
        pltpu.make_async_copy(
            xs.at[par, pl.ds(0, ck)], acc.at[idx2.at[2 * par]],
            sem_s[par]).wait()
    plsc.subcore_barrier()

    def wb_body(k2, carry):
      c8 = sid + _NS * k2

      @pl.when(c8 < n_wb)
      def _():
        rb = c8 * 8
        pltpu.sync_copy(acc.at[pl.ds(rb, 8)],
                        out_hbm.at[pl.ds(nbase + rb, 8), pl.ds(0, hw)])
        pltpu.sync_copy(acc.at[pl.ds(seg + rb, 8)],
                        out_hbm.at[pl.ds(nbase + rb, 8), pl.ds(hw, hw)])
      return carry

    lax.fori_loop(0, _cdiv(n_wb, _NS), wb_body, 0)

  return k(x, row, col, w)


def kernel(x, edge_index, W1, b1, W2, b2):
  n, d = x.shape
  e_total = edge_index.shape[1]
  row = edge_index[0]
  col = edge_index[1]
  a, b = _proj(x, W1, b1)
  logits = _sc_logits(a, b, row, col, W2.reshape(d))
  w = _softmax(logits.reshape(e_total // 128, 128)).reshape(e_total)
  out = _sc_scatter(x, row, col, w)
  return out, w.reshape(e_total, 1)
